# baseline (speedup 1.0000x reference)
import functools

import jax
import jax.numpy as jnp
from jax import lax
from jax.experimental import pallas as pl
from jax.experimental.pallas import tpu as pltpu
from jax.experimental.pallas import tpu_sc as plsc

NC = 2
NS = 16
NW = NC * NS
LANES = 16
D = 128
W = D // 2
WV = W // LANES


def _unpack_row(w_i32):
    v = plsc.bitcast(w_i32, jnp.bfloat16)
    return plsc.unpack(v, format=plsc.PackFormat.INTERLEAVED)


def _make_gather_agg(n_seg, K, G, NB):
    P = n_seg // NW
    CH = P // G
    R = G * K
    assert P * NW == n_seg and CH * G == P and G % 8 == 0 and CH % NB == 0
    RT = R + (LANES if K < LANES else 0)
    mesh = plsc.VectorSubcoreMesh(core_axis_name="c", subcore_axis_name="s",
                                  num_cores=NC)

    @functools.partial(
        pl.kernel,
        out_type=jax.ShapeDtypeStruct((n_seg, D), jnp.float32),
        mesh=mesh,
        scratch_types=[
            pltpu.VMEM((CH, RT), jnp.int32),
            [pltpu.VMEM((R, W), jnp.int32) for _ in range(NB)],
            [pltpu.VMEM((G, D), jnp.float32) for _ in range(NB)],
            pltpu.VMEM((1, W), jnp.int32),
            pltpu.SemaphoreType.DMA,
            [pltpu.SemaphoreType.DMA for _ in range(NB)],
            [pltpu.SemaphoreType.DMA for _ in range(NB)],
        ],
        compiler_params=pltpu.CompilerParams(needs_layout_passes=False,
                                             use_tc_tiling_on_sc=False),
    )
    def gather_agg(tbl_hbm, idx_hbm, out_hbm, idx_v, rows_v, out_v, row0_v,
                   isem, gsem, osem):
        wid = lax.axis_index("s") * NC + lax.axis_index("c")
        pltpu.sync_copy(tbl_hbm.at[pl.ds(0, 1)], row0_v)
        if RT > R:
            for i in range(CH):
                idx_v[i, pl.ds(R, LANES)] = jnp.zeros((LANES,), jnp.int32)
        for i in range(CH):
            pltpu.async_copy(idx_hbm.at[pl.ds((i * NW + wid) * R, R)],
                             idx_v.at[i, pl.ds(0, R)], isem)
        for i in range(CH):
            pltpu.make_async_copy(idx_hbm.at[pl.ds((i * NW + wid) * R, R)],
                                  idx_v.at[i, pl.ds(0, R)], isem).wait()
        row0 = []
        for j in range(WV):
            a, b = _unpack_row(row0_v[0, pl.ds(j * LANES, LANES)])
            row0 += [a, b]
        lane_ok = lax.iota(jnp.int32, LANES) < K

        def start_gather(g, b):
            pltpu.async_copy(tbl_hbm.at[idx_v.at[g, pl.ds(0, R)]],
                             rows_v[b], gsem[b])

        for b in range(NB):
            start_gather(b, b)

        def outer(gg, carry):
            for b in range(NB):
                g = gg * NB + b
                seg0 = (g * NW + wid) * G
                pltpu.make_async_copy(tbl_hbm.at[idx_v.at[g, pl.ds(0, R)]],
                                      rows_v[b], gsem[b]).wait()

                @pl.when(g >= NB)
                def _wait_out():
                    pltpu.make_async_copy(out_v[b],
                                          out_hbm.at[pl.ds(seg0, G)],
                                          osem[b]).wait()

                @plsc.parallel_loop(0, G, unroll=2)
                def seg_body(e):
                    iv = idx_v[g, pl.ds(e * K, LANES)]
                    cnt_i = plsc.all_reduce_population_count((iv > 0) & lane_ok)
                    cnt = cnt_i.astype(jnp.float32)
                    pos = cnt_i > 0
                    scale = jnp.where(pos, 1.0 / jnp.maximum(cnt, 1.0), 1.0 / K)
                    subc = jnp.where(pos, K - cnt, 0.0)
                    rbase = e * K
                    for j in range(WV):
                        wsl = pl.ds(j * LANES, LANES)
                        acc_a, acc_b = _unpack_row(rows_v[b][rbase, wsl])
                        for kk in range(1, K):
                            a, bb = _unpack_row(rows_v[b][rbase + kk, wsl])
                            acc_a = acc_a + a
                            acc_b = acc_b + bb
                        out_v[b][e, pl.ds(j * 32, LANES)] = (
                            acc_a - subc * row0[2 * j]) * scale
                        out_v[b][e, pl.ds(j * 32 + LANES, LANES)] = (
                            acc_b - subc * row0[2 * j + 1]) * scale

                pltpu.async_copy(out_v[b], out_hbm.at[pl.ds(seg0, G)], osem[b])

                @pl.when(g + NB < CH)
                def _next_gather():
                    start_gather(g + NB, b)
            return carry

        lax.fori_loop(0, CH // NB, outer, 0)
        for b in range(NB):
            g = CH - NB + b
            pltpu.make_async_copy(
                out_v[b], out_hbm.at[pl.ds((g * NW + wid) * G, G)],
                osem[b]).wait()

    return gather_agg


def _pack_words(block):
    lo = jnp.concatenate([block[:, 32 * j:32 * j + 16] for j in range(4)],
                         axis=1)
    hi = jnp.concatenate([block[:, 32 * j + 16:32 * j + 32] for j in range(4)],
                         axis=1)
    lo_u = lax.bitcast_convert_type(lo.astype(jnp.bfloat16),
                                    jnp.uint16).astype(jnp.uint32)
    hi_u = lax.bitcast_convert_type(hi.astype(jnp.bfloat16),
                                    jnp.uint16).astype(jnp.uint32)
    return lax.bitcast_convert_type(lo_u | (hi_u << 16), jnp.int32)


def _pack_x(x):
    M = x.shape[0]
    BLK = 2000

    def body(x_ref, o_ref):
        o_ref[...] = _pack_words(x_ref[...])

    return pl.pallas_call(
        body,
        grid=(M // BLK,),
        in_specs=[pl.BlockSpec((BLK, D), lambda i: (i, 0))],
        out_specs=pl.BlockSpec((BLK, W), lambda i: (i, 0)),
        out_shape=jax.ShapeDtypeStruct((M, W), jnp.int32),
    )(x)


def _edge_mlp(agg, w1, w2):
    M = agg.shape[0]
    BLK = 2048

    def body(a_ref, w1_ref, w2_ref, o_ref):
        h = jnp.maximum(
            jnp.dot(a_ref[...], w1_ref[...], preferred_element_type=jnp.float32),
            0.0)
        e1 = jnp.dot(h, w2_ref[...], preferred_element_type=jnp.float32)
        o_ref[...] = _pack_words(e1)

    return pl.pallas_call(
        body,
        grid=(M // BLK,),
        in_specs=[
            pl.BlockSpec((BLK, D), lambda i: (i, 0)),
            pl.BlockSpec((D, D), lambda i: (0, 0)),
            pl.BlockSpec((D, D), lambda i: (0, 0)),
        ],
        out_specs=pl.BlockSpec((BLK, W), lambda i: (i, 0)),
        out_shape=jax.ShapeDtypeStruct((M, W), jnp.int32),
    )(agg, w1, w2)


_EPAD = 20480
_gather_edges = _make_gather_agg(_EPAD, 16, 8, 8)
_UPAD = 51200
_gather_nodes = _make_gather_agg(_UPAD, 8, 16, 10)


def kernel(x, seq, text2emb, useq, data_idx, weight1, weight2, weight3):
    E = seq.shape[0]
    seq_p = jnp.pad(jnp.asarray(seq, jnp.int32), ((0, _EPAD - E), (0, 0)))
    seqf = seq_p.reshape(-1)
    U = useq.shape[0]
    useq_p = jnp.pad(jnp.asarray(useq, jnp.int32), ((0, _UPAD - U), (0, 0)))
    useqf = useq_p.reshape(-1)

    x_i = _pack_x(x)
    agg = _gather_edges(x_i, seqf)
    e1_i = _edge_mlp(agg, weight1, weight2)
    node = _gather_nodes(e1_i, useqf)
    return node[:U]


# speedup vs baseline: 1.0010x; 1.0010x over previous
import functools

import jax
import jax.numpy as jnp
import numpy as np
from jax import lax
from jax.experimental import pallas as pl
from jax.experimental.pallas import tpu as pltpu
from jax.experimental.pallas import tpu_sc as plsc

NC = 2
NS = 16
NW = NC * NS
LANES = 16
D = 128
WV = D // 32

_DEPERM = np.zeros(D, dtype=np.int32)
for _q in range(D):
    _j, _r = _q // 32, _q % 32
    _DEPERM[_q] = 32 * _j + 2 * (_r % 16) + (1 if _r >= 16 else 0)
_INVD = np.argsort(_DEPERM)


def _unpack_row(w_i16):
    v = plsc.bitcast(w_i16, jnp.bfloat16)
    return plsc.unpack(v, format=plsc.PackFormat.INTERLEAVED)


def _make_gather_agg(n_seg, K, G, NB):
    P = n_seg // NW
    CH = P // G
    R = G * K
    assert P * NW == n_seg and CH * G == P and G % 8 == 0 and CH % NB == 0
    RT = R + (LANES if K < LANES else 0)
    mesh = plsc.VectorSubcoreMesh(core_axis_name="c", subcore_axis_name="s",
                                  num_cores=NC)

    @functools.partial(
        pl.kernel,
        out_type=jax.ShapeDtypeStruct((n_seg, D), jnp.float32),
        mesh=mesh,
        scratch_types=[
            pltpu.VMEM((CH, RT), jnp.int32),
            [pltpu.VMEM((R, D), jnp.int16) for _ in range(NB)],
            [pltpu.VMEM((G, D), jnp.float32) for _ in range(NB)],
            pltpu.VMEM((1, D), jnp.int16),
            pltpu.SemaphoreType.DMA,
            [pltpu.SemaphoreType.DMA for _ in range(NB)],
            [pltpu.SemaphoreType.DMA for _ in range(NB)],
        ],
        compiler_params=pltpu.CompilerParams(needs_layout_passes=False,
                                             use_tc_tiling_on_sc=False),
    )
    def gather_agg(tbl_hbm, idx_hbm, out_hbm, idx_v, rows_v, out_v, row0_v,
                   isem, gsem, osem):
        wid = lax.axis_index("s") * NC + lax.axis_index("c")
        pltpu.sync_copy(tbl_hbm.at[pl.ds(0, 1)], row0_v)
        if RT > R:
            for i in range(CH):
                idx_v[i, pl.ds(R, LANES)] = jnp.zeros((LANES,), jnp.int32)
        for i in range(CH):
            pltpu.async_copy(idx_hbm.at[pl.ds((i * NW + wid) * R, R)],
                             idx_v.at[i, pl.ds(0, R)], isem)
        for i in range(CH):
            pltpu.make_async_copy(idx_hbm.at[pl.ds((i * NW + wid) * R, R)],
                                  idx_v.at[i, pl.ds(0, R)], isem).wait()
        row0 = []
        for j in range(WV):
            a, b = _unpack_row(row0_v[0, pl.ds(j * 32, 32)])
            row0 += [a, b]
        lane_ok = lax.iota(jnp.int32, LANES) < K

        def start_gather(g, b):
            pltpu.async_copy(tbl_hbm.at[idx_v.at[g, pl.ds(0, R)]],
                             rows_v[b], gsem[b])

        for b in range(NB):
            start_gather(b, b)

        def outer(gg, carry):
            for b in range(NB):
                g = gg * NB + b
                seg0 = (g * NW + wid) * G
                pltpu.make_async_copy(tbl_hbm.at[idx_v.at[g, pl.ds(0, R)]],
                                      rows_v[b], gsem[b]).wait()

                @pl.when(g >= NB)
                def _wait_out():
                    pltpu.make_async_copy(out_v[b],
                                          out_hbm.at[pl.ds(seg0, G)],
                                          osem[b]).wait()

                @plsc.parallel_loop(0, G, unroll=2)
                def seg_body(e):
                    iv = idx_v[g, pl.ds(e * K, LANES)]
                    cnt_i = plsc.all_reduce_population_count((iv > 0) & lane_ok)
                    cnt = cnt_i.astype(jnp.float32)
                    pos = cnt_i > 0
                    scale = jnp.where(pos, 1.0 / jnp.maximum(cnt, 1.0), 1.0 / K)
                    subc = jnp.where(pos, K - cnt, 0.0)
                    rbase = e * K
                    for j in range(WV):
                        wsl = pl.ds(j * 32, 32)
                        acc_a, acc_b = _unpack_row(rows_v[b][rbase, wsl])
                        for kk in range(1, K):
                            a, bb = _unpack_row(rows_v[b][rbase + kk, wsl])
                            acc_a = acc_a + a
                            acc_b = acc_b + bb
                        out_v[b][e, pl.ds(j * 32, LANES)] = (
                            acc_a - subc * row0[2 * j]) * scale
                        out_v[b][e, pl.ds(j * 32 + LANES, LANES)] = (
                            acc_b - subc * row0[2 * j + 1]) * scale

                pltpu.async_copy(out_v[b], out_hbm.at[pl.ds(seg0, G)], osem[b])

                @pl.when(g + NB < CH)
                def _next_gather():
                    start_gather(g + NB, b)
            return carry

        lax.fori_loop(0, CH // NB, outer, 0)
        for b in range(NB):
            g = CH - NB + b
            pltpu.make_async_copy(
                out_v[b], out_hbm.at[pl.ds((g * NW + wid) * G, G)],
                osem[b]).wait()

    return gather_agg


def _to_bf16_bits(block):
    return lax.bitcast_convert_type(block.astype(jnp.bfloat16), jnp.int16)


def _pack_x(x):
    M = x.shape[0]
    BLK = 2000

    def body(x_ref, o_ref):
        o_ref[...] = _to_bf16_bits(x_ref[...])

    return pl.pallas_call(
        body,
        grid=(M // BLK,),
        in_specs=[pl.BlockSpec((BLK, D), lambda i: (i, 0))],
        out_specs=pl.BlockSpec((BLK, D), lambda i: (i, 0)),
        out_shape=jax.ShapeDtypeStruct((M, D), jnp.int16),
    )(x)


def _edge_mlp(agg, w1p, w2p):
    M = agg.shape[0]
    BLK = 2048

    def body(a_ref, w1_ref, w2_ref, o_ref):
        h = jnp.maximum(
            jnp.dot(a_ref[...], w1_ref[...], preferred_element_type=jnp.float32),
            0.0)
        e1 = jnp.dot(h, w2_ref[...], preferred_element_type=jnp.float32)
        o_ref[...] = _to_bf16_bits(e1)

    return pl.pallas_call(
        body,
        grid=(M // BLK,),
        in_specs=[
            pl.BlockSpec((BLK, D), lambda i: (i, 0)),
            pl.BlockSpec((D, D), lambda i: (0, 0)),
            pl.BlockSpec((D, D), lambda i: (0, 0)),
        ],
        out_specs=pl.BlockSpec((BLK, D), lambda i: (i, 0)),
        out_shape=jax.ShapeDtypeStruct((M, D), jnp.int16),
    )(agg, w1p, w2p)


_EPAD = 20480
_gather_edges = _make_gather_agg(_EPAD, 16, 16, 4)
_UPAD = 51200
_gather_nodes = _make_gather_agg(_UPAD, 8, 40, 4)


def kernel(x, seq, text2emb, useq, data_idx, weight1, weight2, weight3):
    E = seq.shape[0]
    seq_p = jnp.pad(jnp.asarray(seq, jnp.int32), ((0, _EPAD - E), (0, 0)))
    seqf = seq_p.reshape(-1)
    U = useq.shape[0]
    useq_p = jnp.pad(jnp.asarray(useq, jnp.int32), ((0, _UPAD - U), (0, 0)))
    useqf = useq_p.reshape(-1)

    w1p = weight1[_DEPERM, :]
    w2p = weight2[:, _INVD]

    x_i = _pack_x(x)
    agg = _gather_edges(x_i, seqf)
    e1_i = _edge_mlp(agg, w1p, w2p)
    node = _gather_nodes(e1_i, useqf)
    return node[:U]


# speedup vs baseline: 1.0444x; 1.0433x over previous
import functools

import jax
import jax.numpy as jnp
from jax import lax
from jax.experimental import pallas as pl
from jax.experimental.pallas import tpu as pltpu
from jax.experimental.pallas import tpu_sc as plsc

NC = 2
NS = 16
NW = NC * NS
LANES = 16
D = 128
W = D // 2
WV = W // LANES


def _unpack_row(w_i32):
    v = plsc.bitcast(w_i32, jnp.bfloat16)
    return plsc.unpack(v, format=plsc.PackFormat.INTERLEAVED)


def _make_gather_agg(n_seg, K, G, NB):
    P = n_seg // NW
    CH = P // G
    R = G * K
    assert P * NW == n_seg and CH * G == P and G % 8 == 0 and CH % NB == 0
    RT = R + (LANES if K < LANES else 0)
    mesh = plsc.VectorSubcoreMesh(core_axis_name="c", subcore_axis_name="s",
                                  num_cores=NC)

    @functools.partial(
        pl.kernel,
        out_type=jax.ShapeDtypeStruct((n_seg, D), jnp.float32),
        mesh=mesh,
        scratch_types=[
            pltpu.VMEM((CH, RT), jnp.int32),
            [pltpu.VMEM((R, W), jnp.int32) for _ in range(NB)],
            [pltpu.VMEM((G, D), jnp.float32) for _ in range(NB)],
            pltpu.VMEM((1, W), jnp.int32),
            pltpu.SemaphoreType.DMA,
            [pltpu.SemaphoreType.DMA for _ in range(NB)],
            [pltpu.SemaphoreType.DMA for _ in range(NB)],
        ],
        compiler_params=pltpu.CompilerParams(needs_layout_passes=False,
                                             use_tc_tiling_on_sc=False),
    )
    def gather_agg(tbl_hbm, idx_hbm, out_hbm, idx_v, rows_v, out_v, row0_v,
                   isem, gsem, osem):
        wid = lax.axis_index("s") * NC + lax.axis_index("c")
        pltpu.sync_copy(tbl_hbm.at[pl.ds(0, 1)], row0_v)
        if RT > R:
            for i in range(CH):
                idx_v[i, pl.ds(R, LANES)] = jnp.zeros((LANES,), jnp.int32)
        for i in range(CH):
            pltpu.async_copy(idx_hbm.at[pl.ds((i * NW + wid) * R, R)],
                             idx_v.at[i, pl.ds(0, R)], isem)
        for i in range(CH):
            pltpu.make_async_copy(idx_hbm.at[pl.ds((i * NW + wid) * R, R)],
                                  idx_v.at[i, pl.ds(0, R)], isem).wait()
        row0 = []
        for j in range(WV):
            a, b = _unpack_row(row0_v[0, pl.ds(j * LANES, LANES)])
            row0 += [a, b]
        lane_ok = lax.iota(jnp.int32, LANES) < K

        def start_gather(g, b):
            pltpu.async_copy(tbl_hbm.at[idx_v.at[g, pl.ds(0, R)]],
                             rows_v[b], gsem[b])

        for b in range(NB):
            start_gather(b, b)

        def outer(gg, carry):
            for b in range(NB):
                g = gg * NB + b
                seg0 = (g * NW + wid) * G
                pltpu.make_async_copy(tbl_hbm.at[idx_v.at[g, pl.ds(0, R)]],
                                      rows_v[b], gsem[b]).wait()

                @pl.when(g >= NB)
                def _wait_out():
                    pltpu.make_async_copy(out_v[b],
                                          out_hbm.at[pl.ds(seg0, G)],
                                          osem[b]).wait()

                @plsc.parallel_loop(0, G, unroll=2)
                def seg_body(e):
                    iv = idx_v[g, pl.ds(e * K, LANES)]
                    cnt_i = plsc.all_reduce_population_count((iv > 0) & lane_ok)
                    cnt = cnt_i.astype(jnp.float32)
                    pos = cnt_i > 0
                    scale = jnp.where(pos, 1.0 / jnp.maximum(cnt, 1.0), 1.0 / K)
                    subc = jnp.where(pos, K - cnt, 0.0)
                    rbase = e * K
                    for j in range(WV):
                        wsl = pl.ds(j * LANES, LANES)
                        acc_a, acc_b = _unpack_row(rows_v[b][rbase, wsl])
                        for kk in range(1, K):
                            a, bb = _unpack_row(rows_v[b][rbase + kk, wsl])
                            acc_a = acc_a + a
                            acc_b = acc_b + bb
                        out_v[b][e, pl.ds(j * 32, LANES)] = (
                            acc_a - subc * row0[2 * j]) * scale
                        out_v[b][e, pl.ds(j * 32 + LANES, LANES)] = (
                            acc_b - subc * row0[2 * j + 1]) * scale

                pltpu.async_copy(out_v[b], out_hbm.at[pl.ds(seg0, G)], osem[b])

                @pl.when(g + NB < CH)
                def _next_gather():
                    start_gather(g + NB, b)
            return carry

        lax.fori_loop(0, CH // NB, outer, 0)
        for b in range(NB):
            g = CH - NB + b
            pltpu.make_async_copy(
                out_v[b], out_hbm.at[pl.ds((g * NW + wid) * G, G)],
                osem[b]).wait()

    return gather_agg


def _pack_words(block):
    lo = jnp.concatenate([block[:, 32 * j:32 * j + 16] for j in range(4)],
                         axis=1)
    hi = jnp.concatenate([block[:, 32 * j + 16:32 * j + 32] for j in range(4)],
                         axis=1)
    lo_u = lax.bitcast_convert_type(lo.astype(jnp.bfloat16),
                                    jnp.uint16).astype(jnp.uint32)
    hi_u = lax.bitcast_convert_type(hi.astype(jnp.bfloat16),
                                    jnp.uint16).astype(jnp.uint32)
    return lax.bitcast_convert_type(lo_u | (hi_u << 16), jnp.int32)


def _pack_x(x):
    M = x.shape[0]
    BLK = 2000

    def body(x_ref, o_ref):
        o_ref[...] = _pack_words(x_ref[...])

    return pl.pallas_call(
        body,
        grid=(M // BLK,),
        in_specs=[pl.BlockSpec((BLK, D), lambda i: (i, 0))],
        out_specs=pl.BlockSpec((BLK, W), lambda i: (i, 0)),
        out_shape=jax.ShapeDtypeStruct((M, W), jnp.int32),
    )(x)


def _edge_mlp(agg, w1, w2):
    M = agg.shape[0]
    BLK = 2048

    def body(a_ref, w1_ref, w2_ref, o_ref):
        h = jnp.maximum(
            jnp.dot(a_ref[...], w1_ref[...], preferred_element_type=jnp.float32),
            0.0)
        e1 = jnp.dot(h, w2_ref[...], preferred_element_type=jnp.float32)
        o_ref[...] = _pack_words(e1)

    return pl.pallas_call(
        body,
        grid=(M // BLK,),
        in_specs=[
            pl.BlockSpec((BLK, D), lambda i: (i, 0)),
            pl.BlockSpec((D, D), lambda i: (0, 0)),
            pl.BlockSpec((D, D), lambda i: (0, 0)),
        ],
        out_specs=pl.BlockSpec((BLK, W), lambda i: (i, 0)),
        out_shape=jax.ShapeDtypeStruct((M, W), jnp.int32),
    )(agg, w1, w2)


_EPAD = 20480
_gather_edges = _make_gather_agg(_EPAD, 16, 16, 4)
_UPAD = 51200
_gather_nodes = _make_gather_agg(_UPAD, 8, 40, 4)


def kernel(x, seq, text2emb, useq, data_idx, weight1, weight2, weight3):
    E = seq.shape[0]
    seq_p = jnp.pad(jnp.asarray(seq, jnp.int32), ((0, _EPAD - E), (0, 0)))
    seqf = seq_p.reshape(-1)
    U = useq.shape[0]
    useq_p = jnp.pad(jnp.asarray(useq, jnp.int32), ((0, _UPAD - U), (0, 0)))
    useqf = useq_p.reshape(-1)

    x_i = _pack_x(x)
    agg = _gather_edges(x_i, seqf)
    e1_i = _edge_mlp(agg, weight1, weight2)
    node = _gather_nodes(e1_i, useqf)
    return node[:U]
